# baseline (speedup 1.0000x reference)
import functools

import jax
import jax.numpy as jnp
from jax import lax
from jax.experimental import pallas as pl
from jax.experimental.pallas import tpu as pltpu
from jax.experimental.pallas import tpu_sc as plsc

B, C, F = 1024, 50, 1000
NTRIM = 5
KEEP = C - 2 * NTRIM
L = 16
FP = 1008
NC, NS = 2, 16
NW = NC * NS
BPW = B // NW
NFT = FP // L


def _tm_body(x_hbm, out_hbm, xbuf, obuf):
    cid = lax.axis_index("c")
    sid = lax.axis_index("s")
    wid = sid * NC + cid
    base = wid * BPW

    @pl.loop(0, BPW)
    def _batch(i):
        b = base + i
        pltpu.sync_copy(x_hbm.at[b], xbuf.at[:, pl.ds(0, F)])

        @plsc.parallel_loop(0, NFT, unroll=2)
        def _ftile(ft):
            off = pl.multiple_of(ft * L, L)
            neg = jnp.full((L,), -jnp.inf, jnp.float32)
            pos = jnp.full((L,), jnp.inf, jnp.float32)
            top = [neg] * NTRIM
            bot = [pos] * NTRIM
            tot = jnp.zeros((L,), jnp.float32)
            for c in range(C):
                v = xbuf[c, pl.ds(off, L)]
                tot = tot + v
                u = v
                for k in range(NTRIM):
                    hi = jnp.maximum(top[k], u)
                    u = jnp.minimum(top[k], u)
                    top[k] = hi
                w = v
                for k in range(NTRIM):
                    lo = jnp.minimum(bot[k], w)
                    w = jnp.maximum(bot[k], w)
                    bot[k] = lo
            for k in range(NTRIM):
                tot = tot - top[k] - bot[k]
            obuf[pl.ds(off, L)] = tot * (1.0 / KEEP)

        pltpu.sync_copy(obuf.at[pl.ds(0, F)], out_hbm.at[b])


def kernel(x, mask):
    del mask
    mesh = plsc.VectorSubcoreMesh(core_axis_name="c", subcore_axis_name="s")
    tm = pl.kernel(
        _tm_body,
        out_type=jax.ShapeDtypeStruct((B, F), jnp.float32),
        mesh=mesh,
        scratch_types=[
            pltpu.VMEM((C, FP), jnp.float32),
            pltpu.VMEM((FP,), jnp.float32),
        ],
        compiler_params=pltpu.CompilerParams(use_tc_tiling_on_sc=False),
    )
    return tm(x)


# speedup vs baseline: 1.3521x; 1.3521x over previous
import functools

import jax
import jax.numpy as jnp
from jax import lax
from jax.experimental import pallas as pl
from jax.experimental.pallas import tpu as pltpu
from jax.experimental.pallas import tpu_sc as plsc

B, C, F = 1024, 50, 1000
NTRIM = 5
KEEP = C - 2 * NTRIM
L = 16
FP = 1008
NC, NS = 2, 16
NW = NC * NS
NFT = FP // L

B_TC = 512
B_SC = B - B_TC
BPW = B_SC // NW
TC_BLK = 16


def _tm_sc_body(x_hbm, out_hbm, xbuf, obuf):
    cid = lax.axis_index("c")
    sid = lax.axis_index("s")
    wid = sid * NC + cid
    base = wid * BPW

    @pl.loop(0, BPW)
    def _batch(i):
        b = base + i
        pltpu.sync_copy(x_hbm.at[b], xbuf.at[:, pl.ds(0, F)])

        @plsc.parallel_loop(0, NFT)
        def _ftile(ft):
            off = pl.multiple_of(ft * L, L)
            neg = jnp.full((L,), -jnp.inf, jnp.float32)
            pos = jnp.full((L,), jnp.inf, jnp.float32)
            top = [neg] * NTRIM
            bot = [pos] * NTRIM
            tot = jnp.zeros((L,), jnp.float32)
            for c in range(C):
                v = xbuf[c, pl.ds(off, L)]
                tot = tot + v
                u = v
                for k in range(NTRIM):
                    hi = jnp.maximum(top[k], u)
                    u = jnp.minimum(top[k], u)
                    top[k] = hi
                w = v
                for k in range(NTRIM):
                    lo = jnp.minimum(bot[k], w)
                    w = jnp.maximum(bot[k], w)
                    bot[k] = lo
            for k in range(NTRIM):
                tot = tot - top[k] - bot[k]
            obuf[pl.ds(off, L)] = tot * (1.0 / KEEP)

        pltpu.sync_copy(obuf.at[pl.ds(0, F)], out_hbm.at[b])


def _sc_call(x_sc):
    mesh = plsc.VectorSubcoreMesh(core_axis_name="c", subcore_axis_name="s")
    tm = pl.kernel(
        _tm_sc_body,
        out_type=jax.ShapeDtypeStruct((B_SC, F), jnp.float32),
        mesh=mesh,
        scratch_types=[
            pltpu.VMEM((C, FP), jnp.float32),
            pltpu.VMEM((FP,), jnp.float32),
        ],
        compiler_params=pltpu.CompilerParams(use_tc_tiling_on_sc=False),
    )
    return tm(x_sc)


def _tm_tc_kernel(x_ref, o_ref):
    neg = jnp.float32(-jnp.inf)
    pos = jnp.float32(jnp.inf)
    shape = (TC_BLK, F)
    top = [jnp.full(shape, neg)] * NTRIM
    bot = [jnp.full(shape, pos)] * NTRIM
    tot = jnp.zeros(shape, jnp.float32)
    for c in range(C):
        v = x_ref[:, c, :]
        tot = tot + v
        u = v
        for k in range(NTRIM):
            hi = jnp.maximum(top[k], u)
            u = jnp.minimum(top[k], u)
            top[k] = hi
        w = v
        for k in range(NTRIM):
            lo = jnp.minimum(bot[k], w)
            w = jnp.maximum(bot[k], w)
            bot[k] = lo
    for k in range(NTRIM):
        tot = tot - top[k] - bot[k]
    o_ref[...] = tot * (1.0 / KEEP)


def _tc_call(x):
    return pl.pallas_call(
        _tm_tc_kernel,
        grid=(B_TC // TC_BLK,),
        in_specs=[pl.BlockSpec((TC_BLK, C, F), lambda i: (i, 0, 0))],
        out_specs=pl.BlockSpec((TC_BLK, F), lambda i: (i, 0)),
        out_shape=jax.ShapeDtypeStruct((B_TC, F), jnp.float32),
    )(x)


def kernel(x, mask):
    del mask
    out_tc = _tc_call(x)
    out_sc = _sc_call(lax.slice_in_dim(x, B_TC, B, axis=0))
    return jnp.concatenate([out_tc, out_sc], axis=0)


# speedup vs baseline: 1.8696x; 1.3828x over previous
import functools

import jax
import jax.numpy as jnp
from jax import lax
from jax.experimental import pallas as pl
from jax.experimental.pallas import tpu as pltpu
from jax.experimental.pallas import tpu_sc as plsc

B, C, F = 1024, 50, 1000
NTRIM = 5
KEEP = C - 2 * NTRIM
L = 16
CP, FP = 56, 1024
NC, NS = 2, 16
NW = NC * NS
NFT = FP // L - 1

B_TC = 672
B_SC = B - B_TC
BPW = B_SC // NW
TC_BLK = 16


def _tm_sc_body(x_hbm, out_hbm, xbuf, obuf):
    cid = lax.axis_index("c")
    sid = lax.axis_index("s")
    wid = sid * NC + cid
    base = wid * BPW

    @pl.loop(0, BPW)
    def _batch(i):
        b = base + i
        pltpu.sync_copy(x_hbm.at[b, pl.ds(0, C)], xbuf)

        @plsc.parallel_loop(0, NFT)
        def _ftile(ft):
            off = pl.multiple_of(ft * L, L)
            neg = jnp.full((L,), -jnp.inf, jnp.float32)
            pos = jnp.full((L,), jnp.inf, jnp.float32)
            top = [neg] * NTRIM
            bot = [pos] * NTRIM
            tot = jnp.zeros((L,), jnp.float32)
            for c in range(C):
                v = xbuf[c, pl.ds(off, L)]
                tot = tot + v
                u = v
                for k in range(NTRIM):
                    hi = jnp.maximum(top[k], u)
                    u = jnp.minimum(top[k], u)
                    top[k] = hi
                w = v
                for k in range(NTRIM):
                    lo = jnp.minimum(bot[k], w)
                    w = jnp.maximum(bot[k], w)
                    bot[k] = lo
            for k in range(NTRIM):
                tot = tot - top[k] - bot[k]
            obuf[pl.ds(off, L)] = tot * (1.0 / KEEP)

        pltpu.sync_copy(obuf.at[pl.ds(0, F)], out_hbm.at[b])


def _sc_call(x_staged):
    mesh = plsc.VectorSubcoreMesh(core_axis_name="c", subcore_axis_name="s")
    tm = pl.kernel(
        _tm_sc_body,
        out_type=jax.ShapeDtypeStruct((B_SC, F), jnp.float32),
        mesh=mesh,
        scratch_types=[
            pltpu.VMEM((C, FP), jnp.float32),
            pltpu.VMEM((FP,), jnp.float32),
        ],
        compiler_params=pltpu.CompilerParams(use_tc_tiling_on_sc=False),
    )
    return tm(x_staged)


def _stage_kernel(x_ref, o_ref):
    o_ref[:, pl.ds(0, C), pl.ds(0, F)] = x_ref[...]


def _stage_call(x):
    return pl.pallas_call(
        _stage_kernel,
        grid=(B_SC // TC_BLK,),
        in_specs=[pl.BlockSpec((TC_BLK, C, F), lambda i: (i + B_TC // TC_BLK, 0, 0))],
        out_specs=pl.BlockSpec((TC_BLK, CP, FP), lambda i: (i, 0, 0)),
        out_shape=jax.ShapeDtypeStruct((B_SC, CP, FP), jnp.float32),
    )(x)


def _tm_tc_kernel(x_ref, o_ref):
    neg = jnp.float32(-jnp.inf)
    pos = jnp.float32(jnp.inf)
    shape = (TC_BLK, F)
    top = [jnp.full(shape, neg)] * NTRIM
    bot = [jnp.full(shape, pos)] * NTRIM
    tot = jnp.zeros(shape, jnp.float32)
    for c in range(C):
        v = x_ref[:, c, :]
        tot = tot + v
        u = v
        for k in range(NTRIM):
            hi = jnp.maximum(top[k], u)
            u = jnp.minimum(top[k], u)
            top[k] = hi
        w = v
        for k in range(NTRIM):
            lo = jnp.minimum(bot[k], w)
            w = jnp.maximum(bot[k], w)
            bot[k] = lo
    for k in range(NTRIM):
        tot = tot - top[k] - bot[k]
    o_ref[...] = tot * (1.0 / KEEP)


def _tc_call(x):
    return pl.pallas_call(
        _tm_tc_kernel,
        grid=(B_TC // TC_BLK,),
        in_specs=[pl.BlockSpec((TC_BLK, C, F), lambda i: (i, 0, 0))],
        out_specs=pl.BlockSpec((TC_BLK, F), lambda i: (i, 0)),
        out_shape=jax.ShapeDtypeStruct((B_TC, F), jnp.float32),
    )(x)


def kernel(x, mask):
    del mask
    x_staged = _stage_call(x)
    out_sc = _sc_call(x_staged)
    out_tc = _tc_call(x)
    return jnp.concatenate([out_tc, out_sc], axis=0)


# speedup vs baseline: 3.0358x; 1.6237x over previous
import functools

import jax
import jax.numpy as jnp
from jax import lax
from jax.experimental import pallas as pl
from jax.experimental.pallas import tpu as pltpu
from jax.experimental.pallas import tpu_sc as plsc

B, C, F = 1024, 50, 1000
NTRIM = 5
KEEP = C - 2 * NTRIM
L = 16
NC, NS = 2, 16
NW = NC * NS

F_SC = 296
F_TC = F - F_SC
NSB = 8
FPW = F_SC // 4
FC = 8
NCHUNK = (FPW + FC - 1) // FC
TC_FB = 8


def _tm_sc_body(x_hbm, out_hbm, xbuf, obuf):
    cid = lax.axis_index("c")
    sid = lax.axis_index("s")
    wid = sid * NC + cid
    s = lax.rem(wid, NSB)
    q = wid // NSB
    fbase = q * FPW

    @pl.loop(0, NCHUNK)
    def _chunk(ci):
        f0 = fbase + jnp.minimum(ci * FC, FPW - FC)
        pltpu.sync_copy(x_hbm.at[:, pl.ds(f0, FC), s], xbuf)

        @plsc.parallel_loop(0, FC * (128 // L))
        def _vec(t):
            f = t // (128 // L)
            l0 = pl.multiple_of(lax.rem(t, 128 // L) * L, L)
            if True:
                neg = jnp.full((L,), -jnp.inf, jnp.float32)
                pos = jnp.full((L,), jnp.inf, jnp.float32)
                top = [neg] * NTRIM
                bot = [pos] * NTRIM
                tot = jnp.zeros((L,), jnp.float32)
                for c in range(C):
                    v = xbuf[c, f, pl.ds(l0, L)]
                    tot = tot + v
                    u = v
                    for k in range(NTRIM):
                        hi = jnp.maximum(top[k], u)
                        u = jnp.minimum(top[k], u)
                        top[k] = hi
                    w = v
                    for k in range(NTRIM):
                        lo = jnp.minimum(bot[k], w)
                        w = jnp.maximum(bot[k], w)
                        bot[k] = lo
                for k in range(NTRIM):
                    tot = tot - top[k] - bot[k]
                obuf[f, pl.ds(l0, L)] = tot * (1.0 / KEEP)

        pltpu.sync_copy(obuf, out_hbm.at[pl.ds(f0, FC), s])


def _sc_call(xt4):
    mesh = plsc.VectorSubcoreMesh(core_axis_name="c", subcore_axis_name="s")
    tm = pl.kernel(
        _tm_sc_body,
        out_type=jax.ShapeDtypeStruct((F_SC, NSB, 128), jnp.float32),
        mesh=mesh,
        scratch_types=[
            pltpu.VMEM((C, FC, 128), jnp.float32),
            pltpu.VMEM((FC, 128), jnp.float32),
        ],
        compiler_params=pltpu.CompilerParams(use_tc_tiling_on_sc=False),
    )
    return tm(xt4)


def _tm_tc_kernel(x_ref, o_ref):
    neg = jnp.float32(-jnp.inf)
    pos = jnp.float32(jnp.inf)
    shape = (TC_FB, B)
    top = [jnp.full(shape, neg)] * NTRIM
    bot = [jnp.full(shape, pos)] * NTRIM
    tot = jnp.zeros(shape, jnp.float32)
    for c in range(C):
        v = x_ref[c]
        tot = tot + v
        u = v
        for k in range(NTRIM):
            hi = jnp.maximum(top[k], u)
            u = jnp.minimum(top[k], u)
            top[k] = hi
        w = v
        for k in range(NTRIM):
            lo = jnp.minimum(bot[k], w)
            w = jnp.maximum(bot[k], w)
            bot[k] = lo
    for k in range(NTRIM):
        tot = tot - top[k] - bot[k]
    o_ref[...] = tot * (1.0 / KEEP)


def _tc_call(xt):
    return pl.pallas_call(
        _tm_tc_kernel,
        grid=(F_TC // TC_FB,),
        in_specs=[pl.BlockSpec((C, TC_FB, B), lambda i: (0, i + F_SC // TC_FB, 0))],
        out_specs=pl.BlockSpec((TC_FB, B), lambda i: (i, 0)),
        out_shape=jax.ShapeDtypeStruct((F_TC, B), jnp.float32),
    )(xt)


def kernel(x, mask):
    del mask
    xt = jnp.transpose(x, (1, 2, 0))
    out_sc = _sc_call(xt.reshape(C, F, NSB, 128)).reshape(F_SC, B)
    out_tc = _tc_call(xt)
    out_t = jnp.concatenate([out_sc, out_tc], axis=0)
    return jnp.transpose(out_t, (1, 0))


# speedup vs baseline: 4.0507x; 1.3343x over previous
import functools

import jax
import jax.numpy as jnp
from jax import lax
from jax.experimental import pallas as pl
from jax.experimental.pallas import tpu as pltpu
from jax.experimental.pallas import tpu_sc as plsc

B, C, F = 1024, 50, 1000
NTRIM = 5
KEEP = C - 2 * NTRIM
L = 16
NC, NS = 2, 16
NW = NC * NS

F_SC = 272
F_TC = F - F_SC
NSB = 8
FPW = F_SC // 4
FC = 8
NCHUNK = (FPW + FC - 1) // FC
TC_FB = 8


def _tm_sc_body(x_hbm, out_hbm, xbuf, obuf):
    cid = lax.axis_index("c")
    sid = lax.axis_index("s")
    wid = sid * NC + cid
    s = lax.rem(wid, NSB)
    q = wid // NSB
    fbase = q * FPW

    @pl.loop(0, NCHUNK)
    def _chunk(ci):
        f0 = fbase + jnp.minimum(ci * FC, FPW - FC)
        pltpu.sync_copy(x_hbm.at[:, pl.ds(f0, FC), s], xbuf)

        @plsc.parallel_loop(0, FC * (128 // L))
        def _vec(t):
            f = t // (128 // L)
            l0 = pl.multiple_of(lax.rem(t, 128 // L) * L, L)
            if True:
                neg = jnp.full((L,), -jnp.inf, jnp.float32)
                pos = jnp.full((L,), jnp.inf, jnp.float32)
                top = [neg] * NTRIM
                bot = [pos] * NTRIM
                tot = jnp.zeros((L,), jnp.float32)
                for c in range(C):
                    v = xbuf[c, f, pl.ds(l0, L)]
                    tot = tot + v
                    u = v
                    for k in range(NTRIM):
                        hi = jnp.maximum(top[k], u)
                        u = jnp.minimum(top[k], u)
                        top[k] = hi
                    w = v
                    for k in range(NTRIM):
                        lo = jnp.minimum(bot[k], w)
                        w = jnp.maximum(bot[k], w)
                        bot[k] = lo
                for k in range(NTRIM):
                    tot = tot - top[k] - bot[k]
                obuf[f, pl.ds(l0, L)] = tot * (1.0 / KEEP)

        pltpu.sync_copy(obuf, out_hbm.at[pl.ds(f0, FC), s])


def _sc_call(xt4):
    mesh = plsc.VectorSubcoreMesh(core_axis_name="c", subcore_axis_name="s")
    tm = pl.kernel(
        _tm_sc_body,
        out_type=jax.ShapeDtypeStruct((F_SC, NSB, 128), jnp.float32),
        mesh=mesh,
        scratch_types=[
            pltpu.VMEM((C, FC, 128), jnp.float32),
            pltpu.VMEM((FC, 128), jnp.float32),
        ],
        compiler_params=pltpu.CompilerParams(use_tc_tiling_on_sc=False),
    )
    return tm(xt4)


def _tm_tc_kernel(x_ref, o_ref):
    neg = jnp.float32(-jnp.inf)
    pos = jnp.float32(jnp.inf)
    shape = (TC_FB, B)
    top = [jnp.full(shape, neg)] * NTRIM
    bot = [jnp.full(shape, pos)] * NTRIM
    tot = jnp.zeros(shape, jnp.float32)
    for c in range(C):
        v = x_ref[c]
        tot = tot + v
        u = v
        for k in range(NTRIM):
            hi = jnp.maximum(top[k], u)
            u = jnp.minimum(top[k], u)
            top[k] = hi
        w = v
        for k in range(NTRIM):
            lo = jnp.minimum(bot[k], w)
            w = jnp.maximum(bot[k], w)
            bot[k] = lo
    for k in range(NTRIM):
        tot = tot - top[k] - bot[k]
    o_ref[...] = tot * (1.0 / KEEP)


def _tc_call(xt):
    return pl.pallas_call(
        _tm_tc_kernel,
        grid=(F_TC // TC_FB,),
        in_specs=[pl.BlockSpec((C, TC_FB, B), lambda i: (0, i + F_SC // TC_FB, 0))],
        out_specs=pl.BlockSpec((TC_FB, B), lambda i: (i, 0)),
        out_shape=jax.ShapeDtypeStruct((F_TC, B), jnp.float32),
    )(xt)


def kernel(x, mask):
    del mask
    xt = jnp.transpose(x, (1, 2, 0))
    xt_sc = lax.slice_in_dim(xt, 0, F_SC, axis=1)
    out_sc = _sc_call(xt_sc.reshape(C, F_SC, NSB, 128)).reshape(F_SC, B)
    out_tc = _tc_call(xt)
    out_t = jnp.concatenate([out_sc, out_tc], axis=0)
    return jnp.transpose(out_t, (1, 0))


# speedup vs baseline: 4.4536x; 1.0995x over previous
import functools

import jax
import jax.numpy as jnp
from jax import lax
from jax.experimental import pallas as pl
from jax.experimental.pallas import tpu as pltpu
from jax.experimental.pallas import tpu_sc as plsc

B, C, F = 1024, 50, 1000
NTRIM = 5
KEEP = C - 2 * NTRIM
L = 16
NC, NS = 2, 16
NW = NC * NS

F_SC = 248
F_TC = F - F_SC
NSB = 8
FPW = F_SC // 4
FC = 8
NCHUNK = (FPW + FC - 1) // FC
TC_FB = 8


def _tm_sc_body(x_hbm, out_hbm, xbuf, obuf):
    cid = lax.axis_index("c")
    sid = lax.axis_index("s")
    wid = sid * NC + cid
    s = lax.rem(wid, NSB)
    q = wid // NSB
    fbase = q * FPW

    @pl.loop(0, NCHUNK)
    def _chunk(ci):
        f0 = fbase + jnp.minimum(ci * FC, FPW - FC)
        pltpu.sync_copy(x_hbm.at[:, pl.ds(f0, FC), s], xbuf)

        @plsc.parallel_loop(0, FC * (128 // L))
        def _vec(t):
            f = t // (128 // L)
            l0 = pl.multiple_of(lax.rem(t, 128 // L) * L, L)
            if True:
                neg = jnp.full((L,), -jnp.inf, jnp.float32)
                pos = jnp.full((L,), jnp.inf, jnp.float32)
                top = [neg] * NTRIM
                bot = [pos] * NTRIM
                tot = jnp.zeros((L,), jnp.float32)
                for c in range(C):
                    v = xbuf[c, f, pl.ds(l0, L)]
                    tot = tot + v
                    u = v
                    for k in range(NTRIM):
                        hi = jnp.maximum(top[k], u)
                        u = jnp.minimum(top[k], u)
                        top[k] = hi
                    w = v
                    for k in range(NTRIM):
                        lo = jnp.minimum(bot[k], w)
                        w = jnp.maximum(bot[k], w)
                        bot[k] = lo
                for k in range(NTRIM):
                    tot = tot - top[k] - bot[k]
                obuf[f, pl.ds(l0, L)] = tot * (1.0 / KEEP)

        pltpu.sync_copy(obuf, out_hbm.at[pl.ds(f0, FC), s])


def _sc_call(xt4):
    mesh = plsc.VectorSubcoreMesh(core_axis_name="c", subcore_axis_name="s")
    tm = pl.kernel(
        _tm_sc_body,
        out_type=jax.ShapeDtypeStruct((F_SC, NSB, 128), jnp.float32),
        mesh=mesh,
        scratch_types=[
            pltpu.VMEM((C, FC, 128), jnp.float32),
            pltpu.VMEM((FC, 128), jnp.float32),
        ],
        compiler_params=pltpu.CompilerParams(use_tc_tiling_on_sc=False),
    )
    return tm(xt4)


def _tm_tc_kernel(x_ref, o_ref):
    neg = jnp.float32(-jnp.inf)
    pos = jnp.float32(jnp.inf)
    shape = (TC_FB, B)
    top = [jnp.full(shape, neg)] * NTRIM
    bot = [jnp.full(shape, pos)] * NTRIM
    tot = jnp.zeros(shape, jnp.float32)
    for c in range(C):
        v = x_ref[c]
        tot = tot + v
        u = v
        for k in range(NTRIM):
            hi = jnp.maximum(top[k], u)
            u = jnp.minimum(top[k], u)
            top[k] = hi
        w = v
        for k in range(NTRIM):
            lo = jnp.minimum(bot[k], w)
            w = jnp.maximum(bot[k], w)
            bot[k] = lo
    for k in range(NTRIM):
        tot = tot - top[k] - bot[k]
    o_ref[...] = tot * (1.0 / KEEP)


def _tc_call(xt):
    return pl.pallas_call(
        _tm_tc_kernel,
        grid=(F_TC // TC_FB,),
        in_specs=[pl.BlockSpec((C, TC_FB, B), lambda i: (0, i + F_SC // TC_FB, 0))],
        out_specs=pl.BlockSpec((TC_FB, B), lambda i: (i, 0)),
        out_shape=jax.ShapeDtypeStruct((F_TC, B), jnp.float32),
    )(xt)


def kernel(x, mask):
    del mask
    xt = jnp.transpose(x, (1, 2, 0))
    xt_sc = lax.slice_in_dim(xt, 0, F_SC, axis=1)
    out_sc = _sc_call(xt_sc.reshape(C, F_SC, NSB, 128)).reshape(F_SC, B)
    out_tc = _tc_call(xt)
    out_t = jnp.concatenate([out_sc, out_tc], axis=0)
    return jnp.transpose(out_t, (1, 0))


# speedup vs baseline: 4.4561x; 1.0006x over previous
import jax
import jax.numpy as jnp
from jax import lax
from jax.experimental import pallas as pl
from jax.experimental.pallas import tpu as pltpu
from jax.experimental.pallas import tpu_sc as plsc

B, C, F = 1024, 50, 1000
NTRIM = 5
KEEP = C - 2 * NTRIM
L = 16
NC, NS = 2, 16
NW = NC * NS

F_SC = 248
F_TC = F - F_SC
NSB = 8
FPW = F_SC // 4
FC = 8
NCHUNK = (FPW + FC - 1) // FC
TC_FB = 8


def _tm_sc_body(x_hbm, out_hbm, xbuf, obuf):
    cid = lax.axis_index("c")
    sid = lax.axis_index("s")
    wid = sid * NC + cid
    s = lax.rem(wid, NSB)
    q = wid // NSB
    fbase = q * FPW

    @pl.loop(0, NCHUNK)
    def _chunk(ci):
        f0 = fbase + jnp.minimum(ci * FC, FPW - FC)
        pltpu.sync_copy(x_hbm.at[:, pl.ds(f0, FC), s], xbuf)

        @plsc.parallel_loop(0, FC * (128 // L))
        def _vec(t):
            f = t // (128 // L)
            l0 = pl.multiple_of(lax.rem(t, 128 // L) * L, L)
            neg = jnp.full((L,), -jnp.inf, jnp.float32)
            pos = jnp.full((L,), jnp.inf, jnp.float32)
            top = [neg] * NTRIM
            bot = [pos] * NTRIM
            tot = jnp.zeros((L,), jnp.float32)
            for c in range(C):
                v = xbuf[c, f, pl.ds(l0, L)]
                tot = tot + v
                u = v
                for k in range(NTRIM):
                    hi = jnp.maximum(top[k], u)
                    u = jnp.minimum(top[k], u)
                    top[k] = hi
                w = v
                for k in range(NTRIM):
                    lo = jnp.minimum(bot[k], w)
                    w = jnp.maximum(bot[k], w)
                    bot[k] = lo
            for k in range(NTRIM):
                tot = tot - top[k] - bot[k]
            obuf[f, pl.ds(l0, L)] = tot * (1.0 / KEEP)

        pltpu.sync_copy(obuf, out_hbm.at[pl.ds(f0, FC), s])


def _sc_call(xt4):
    mesh = plsc.VectorSubcoreMesh(core_axis_name="c", subcore_axis_name="s")
    tm = pl.kernel(
        _tm_sc_body,
        out_type=jax.ShapeDtypeStruct((F_SC, NSB, 128), jnp.float32),
        mesh=mesh,
        scratch_types=[
            pltpu.VMEM((C, FC, 128), jnp.float32),
            pltpu.VMEM((FC, 128), jnp.float32),
        ],
        compiler_params=pltpu.CompilerParams(use_tc_tiling_on_sc=False),
    )
    return tm(xt4)


def _tm_tc_kernel(x_ref, o_ref):
    neg = jnp.float32(-jnp.inf)
    pos = jnp.float32(jnp.inf)
    shape = (TC_FB, B)
    top = [jnp.full(shape, neg)] * NTRIM
    bot = [jnp.full(shape, pos)] * NTRIM
    tot = jnp.zeros(shape, jnp.float32)
    for c in range(C):
        v = x_ref[c]
        tot = tot + v
        u = v
        for k in range(NTRIM):
            hi = jnp.maximum(top[k], u)
            u = jnp.minimum(top[k], u)
            top[k] = hi
        w = v
        for k in range(NTRIM):
            lo = jnp.minimum(bot[k], w)
            w = jnp.maximum(bot[k], w)
            bot[k] = lo
    for k in range(NTRIM):
        tot = tot - top[k] - bot[k]
    o_ref[...] = tot * (1.0 / KEEP)


def _tc_call(xt):
    return pl.pallas_call(
        _tm_tc_kernel,
        grid=(F_TC // TC_FB,),
        in_specs=[pl.BlockSpec((C, TC_FB, B), lambda i: (0, i + F_SC // TC_FB, 0))],
        out_specs=pl.BlockSpec((TC_FB, B), lambda i: (i, 0)),
        out_shape=jax.ShapeDtypeStruct((F_TC, B), jnp.float32),
    )(xt)


def kernel(x, mask):
    del mask
    xt = jnp.transpose(x, (1, 2, 0))
    xt_sc = lax.slice_in_dim(xt, 0, F_SC, axis=1)
    out_sc = _sc_call(xt_sc.reshape(C, F_SC, NSB, 128)).reshape(F_SC, B)
    out_tc = _tc_call(xt)
    out_t = jnp.concatenate([out_sc, out_tc], axis=0)
    return jnp.transpose(out_t, (1, 0))
